# baseline (speedup 1.0000x reference)
import jax
import jax.numpy as jnp
from jax.experimental import pallas as pl

_ALPHA = 4
_LANES = 128


def kernel(frames):
    B, T, C, H, W = frames.shape
    S = T // _ALPHA
    F = C * H * W
    sub = F // _LANES

    def in_map(b, j):
        return (b, (j * (T - 1)) // (S - 1), 0, 0)

    def body(in_ref, out_ref):
        out_ref[...] = in_ref[...]

    flat = frames.reshape(B, T, sub, _LANES)
    slow = pl.pallas_call(
        body,
        grid=(B, S),
        in_specs=[pl.BlockSpec((1, 1, sub, _LANES), in_map)],
        out_specs=pl.BlockSpec((1, 1, sub, _LANES), lambda b, j: (b, j, 0, 0)),
        out_shape=jax.ShapeDtypeStruct((B, S, sub, _LANES), frames.dtype),
    )(flat)
    return slow.reshape(B, S, C, H, W), frames
